# Initial kernel scaffold; baseline (speedup 1.0000x reference)
import jax
import jax.numpy as jnp
from jax.experimental import pallas as pl


def kernel(pcd_pos, pcd_rgb, pcd_mask, W1, b1, g1, be1, W2, b2, W3, b3, g3, be3, W4, b4):
    raise NotImplementedError("write your pallas kernel here")


# speedup vs baseline: 3.0988x; 3.0988x over previous
import functools

import jax
import jax.numpy as jnp
from jax import lax
from jax.experimental import pallas as pl
from jax.experimental.pallas import tpu as pltpu
from jax.experimental.pallas import tpu_sc as plsc

B = 8
H_IMG = 224
W_IMG = 224
N = H_IMG * W_IMG
R = 392
LANES = 128
G = 64
M = 32
IN_C = 6
BG = B * G
HID = 384
EPS = 1e-5
BIG = 3.0e38


def _fps_body(xyz_ref, cidx_ref, dist_ref):
    rowi = lax.broadcasted_iota(jnp.int32, (R, LANES), 0)
    lanei = lax.broadcasted_iota(jnp.int32, (R, LANES), 1)
    pid = rowi * LANES + lanei
    lane1 = lax.broadcasted_iota(jnp.int32, (1, LANES), 1)
    giota = lax.broadcasted_iota(jnp.int32, (1, G), 1)
    dist_ref[...] = jnp.full((R, LANES), 1e10, jnp.float32)

    def step(k, carry):
        far, civ = carry
        r = far // LANES
        c = far % LANES
        cm = lane1 == c
        d = jnp.zeros((R, LANES), jnp.float32)
        for ch in range(3):
            row = xyz_ref[0, ch, pl.ds(r, 1), :]
            cch = jnp.sum(jnp.where(cm, row, 0.0))
            diff = xyz_ref[0, ch] - cch
            d = d + diff * diff
        nd = jnp.minimum(dist_ref[...], d)
        dist_ref[...] = nd
        civ = jnp.where(giota == k, far, civ)
        mval = jnp.max(nd)
        nxt = jnp.min(jnp.where(nd == mval, pid, N)).astype(jnp.int32)
        return nxt, civ

    _, civ = lax.fori_loop(0, G, step, (jnp.int32(0), jnp.zeros((1, G), jnp.int32)))
    cidx_ref[0] = civ


def _run_fps(xyz4):
    return pl.pallas_call(
        _fps_body,
        grid=(B,),
        in_specs=[pl.BlockSpec((1, 3, R, LANES), lambda b: (b, 0, 0, 0))],
        out_specs=pl.BlockSpec((1, 1, G), lambda b: (b, 0, 0)),
        out_shape=jax.ShapeDtypeStruct((B, 1, G), jnp.int32),
        scratch_shapes=[pltpu.VMEM((R, LANES), jnp.float32)],
    )(xyz4)


def _knn_body(xyz_ref, cidx_ref, knn_ref, cen_ref):
    rowi = lax.broadcasted_iota(jnp.int32, (R, LANES), 0)
    lanei = lax.broadcasted_iota(jnp.int32, (R, LANES), 1)
    pid = rowi * LANES + lanei
    lane1 = lax.broadcasted_iota(jnp.int32, (1, LANES), 1)
    giota = lax.broadcasted_iota(jnp.int32, (1, G), 1)
    miota = lax.broadcasted_iota(jnp.int32, (1, M), 1)

    x0 = xyz_ref[0, 0]
    x1 = xyz_ref[0, 1]
    x2 = xyz_ref[0, 2]
    psq = x0 * x0 + x1 * x1 + x2 * x2
    x0b = x0.astype(jnp.bfloat16).astype(jnp.float32)
    x1b = x1.astype(jnp.bfloat16).astype(jnp.float32)
    x2b = x2.astype(jnp.bfloat16).astype(jnp.float32)

    def g_body(g, _):
        cidx_g = jnp.sum(jnp.where(giota == g, cidx_ref[0], 0))
        r = cidx_g // LANES
        c = cidx_g % LANES
        cm = lane1 == c
        cs = []
        for ch in range(3):
            row = xyz_ref[0, ch, pl.ds(r, 1), :]
            cs.append(jnp.sum(jnp.where(cm, row, 0.0)))
        c0, c1, c2 = cs
        c0b = c0.astype(jnp.bfloat16).astype(jnp.float32)
        c1b = c1.astype(jnp.bfloat16).astype(jnp.float32)
        c2b = c2.astype(jnp.bfloat16).astype(jnp.float32)
        dot = c0b * x0b + c1b * x1b + c2b * x2b
        csq = c0 * c0 + c1 * c1 + c2 * c2
        d2 = (csq + psq) - 2.0 * dot

        def k_body(k, carry):
            d, rowv = carry
            mv = jnp.min(d)
            pk = jnp.min(jnp.where(d == mv, pid, N)).astype(jnp.int32)
            rowv = jnp.where(miota == k, pk, rowv)
            d = jnp.where(pid == pk, BIG, d)
            return d, rowv

        _, rowv = lax.fori_loop(0, M, k_body,
                                (d2, jnp.zeros((1, M), jnp.int32)))
        knn_ref[0, pl.ds(g, 1), :] = rowv
        cen_ref[0, pl.ds(g, 1), :] = jnp.concatenate(
            [c0.reshape(1, 1), c1.reshape(1, 1), c2.reshape(1, 1)], axis=1)
        return 0

    lax.fori_loop(0, G, g_body, 0)


def _run_knn(xyz4, cidx):
    return pl.pallas_call(
        _knn_body,
        grid=(B,),
        in_specs=[
            pl.BlockSpec((1, 3, R, LANES), lambda b: (b, 0, 0, 0)),
            pl.BlockSpec((1, 1, G), lambda b: (b, 0, 0)),
        ],
        out_specs=[
            pl.BlockSpec((1, G, M), lambda b: (b, 0, 0)),
            pl.BlockSpec((1, G, 3), lambda b: (b, 0, 0)),
        ],
        out_shape=[
            jax.ShapeDtypeStruct((B, G, M), jnp.int32),
            jax.ShapeDtypeStruct((B, G, 3), jnp.float32),
        ],
    )(xyz4, cidx)


SC_CHUNK = BG * IN_C
SC_T = 24
BP = B * 3 * N


def _sc_gather_body(table_hbm, knnt_hbm, out_hbm, knn_v, idx_v, rows_v, sem):
    info = plsc.get_sparse_core_info()
    nc = info.num_cores
    wid = lax.axis_index("s") * nc + lax.axis_index("c")
    pltpu.sync_copy(knnt_hbm.at[pl.ds(wid, 1)], knn_v)
    for t in range(SC_T):
        ch = t // 4
        q = t % 4
        for s in range(8):
            bg0 = q * 128 + s * 16
            b = bg0 // G
            if ch < 3:
                base = (b * 3 + ch) * N
            else:
                base = BP + (b * 3 + (ch - 3)) * N
            kv = knn_v[0, pl.ds(bg0, 16)]
            idx_v[t, pl.ds(s * 16, 16)] = kv + base
    cps = []
    for t in range(SC_T):
        cps.append(pltpu.async_copy(table_hbm.at[idx_v.at[t]], rows_v.at[t], sem))
    for cp in cps:
        cp.wait()
    pltpu.sync_copy(rows_v, out_hbm.at[wid])


def _run_sc_gather(table, knn_t):
    mesh = plsc.VectorSubcoreMesh(core_axis_name="c", subcore_axis_name="s")
    k = functools.partial(
        pl.kernel,
        mesh=mesh,
        out_type=jax.ShapeDtypeStruct((32, SC_T, 128), jnp.float32),
        scratch_types=[
            pltpu.VMEM((1, BG), jnp.int32),
            pltpu.VMEM((SC_T, 128), jnp.int32),
            pltpu.VMEM((SC_T, 128), jnp.float32),
            pltpu.SemaphoreType.DMA,
        ],
    )(_sc_gather_body)
    return k(table, knn_t)


def _enc_body(x_ref, cen_ref, w1_ref, b1_ref, g1_ref, be1_ref, w2_ref, b2_ref,
              w3a_ref, w3b_ref, b3_ref, g3_ref, be3_ref, w4_ref, b4_ref,
              out_ref, f2_ref):
    w1 = w1_ref[...]
    npts = jnp.float32(M * BG)
    dn_t = (((0,), (0,)), ((), ()))

    def a_m(m):
        xm = x_ref[pl.ds(m, 1)][0] - cen_ref[...]
        return lax.dot_general(xm, w1, dn_t,
                               preferred_element_type=jnp.float32) + b1_ref[...]

    s1 = lax.fori_loop(
        0, M, lambda m, s: s + jnp.sum(a_m(m), axis=0, keepdims=True),
        jnp.zeros((1, 128), jnp.float32))
    mu1 = s1 / npts

    def q1_body(m, q):
        d = a_m(m) - mu1
        return q + jnp.sum(d * d, axis=0, keepdims=True)

    q1 = lax.fori_loop(0, M, q1_body, jnp.zeros((1, 128), jnp.float32))
    inv1 = g1_ref[...] / jnp.sqrt(q1 / npts + EPS)

    w2 = w2_ref[...]

    def l2_body(m, fg):
        h = jnp.maximum((a_m(m) - mu1) * inv1 + be1_ref[...], 0.0)
        f2 = jnp.dot(h, w2, preferred_element_type=jnp.float32) + b2_ref[...]
        f2_ref[pl.ds(m, 1)] = f2[None]
        return jnp.maximum(fg, f2)

    fg = lax.fori_loop(0, M, l2_body, jnp.full((BG, 256), -BIG, jnp.float32))

    gpart = jnp.dot(fg, w3a_ref[...], preferred_element_type=jnp.float32) + b3_ref[...]
    w3b = w3b_ref[...]

    def f3_m(m):
        return gpart + jnp.dot(f2_ref[pl.ds(m, 1)][0], w3b,
                               preferred_element_type=jnp.float32)

    s3 = lax.fori_loop(
        0, M, lambda m, s: s + jnp.sum(f3_m(m), axis=0, keepdims=True),
        jnp.zeros((1, 512), jnp.float32))
    mu3 = s3 / npts

    def q3_body(m, q):
        d = f3_m(m) - mu3
        return q + jnp.sum(d * d, axis=0, keepdims=True)

    q3 = lax.fori_loop(0, M, q3_body, jnp.zeros((1, 512), jnp.float32))
    inv3 = g3_ref[...] / jnp.sqrt(q3 / npts + EPS)

    w4 = w4_ref[...]

    def l4_body(m, acc):
        h = jnp.maximum((f3_m(m) - mu3) * inv3 + be3_ref[...], 0.0)
        f4 = jnp.dot(h, w4, preferred_element_type=jnp.float32) + b4_ref[...]
        return jnp.maximum(acc, f4)

    acc = lax.fori_loop(0, M, l4_body, jnp.full((BG, HID), -BIG, jnp.float32))
    out_ref[...] = acc


def _run_encoder(x, cen6, w1t, b1, g1, be1, w2t, b2, w3at, w3bt, b3, g3, be3,
                 w4t, b4):
    return pl.pallas_call(
        _enc_body,
        out_shape=jax.ShapeDtypeStruct((BG, HID), jnp.float32),
        scratch_shapes=[
            pltpu.VMEM((M, BG, 256), jnp.float32),
        ],
    )(x, cen6, w1t, b1, g1, be1, w2t, b2, w3at, w3bt, b3, g3, be3, w4t, b4)


def kernel(pcd_pos, pcd_rgb, pcd_mask, W1, b1, g1, be1, W2, b2, W3, b3, g3,
           be3, W4, b4):
    del pcd_mask
    xyz4 = pcd_pos.reshape(B, 3, R, LANES)
    cidx = _run_fps(xyz4)
    knn, centers = _run_knn(xyz4, cidx)

    table = jnp.concatenate([pcd_pos.reshape(-1), pcd_rgb.reshape(-1)])
    knn_t = knn.reshape(BG, M).T
    xg = _run_sc_gather(table, knn_t)
    x = xg.reshape(M, IN_C, BG)
    cen_t = jnp.pad(centers.reshape(BG, 3).T, ((0, 3), (0, 0)))

    out = _run_encoder(
        x, cen_t,
        W1.T,
        b1.reshape(1, 128), g1.reshape(1, 128), be1.reshape(1, 128),
        W2.T, b2.reshape(1, 256),
        W3[:, :256].T, W3[:, 256:].T, b3.reshape(1, 512),
        g3.reshape(1, 512), be3.reshape(1, 512),
        W4.T, b4.reshape(1, HID),
    )
    return out.reshape(B, G, HID)


# speedup vs baseline: 9.1687x; 2.9588x over previous
import functools

import jax
import jax.numpy as jnp
from jax import lax
from jax.experimental import pallas as pl
from jax.experimental.pallas import tpu as pltpu
from jax.experimental.pallas import tpu_sc as plsc

B = 8
H_IMG = 224
W_IMG = 224
N = H_IMG * W_IMG
R = 392
LANES = 128
G = 64
M = 32
IN_C = 6
BG = B * G
HID = 384
EPS = 1e-5
BIG = 3.0e38


FPS_BH = 4


def _fps_body(xyz_ref, cidx_ref, dist_ref):
    rowi = lax.broadcasted_iota(jnp.int32, (R, LANES), 0)
    lanei = lax.broadcasted_iota(jnp.int32, (R, LANES), 1)
    pid = rowi * LANES + lanei
    pid3 = pid[None]
    lane1 = lax.broadcasted_iota(jnp.int32, (1, LANES), 1)
    biota3 = lax.broadcasted_iota(jnp.int32, (FPS_BH, 1, 1), 0)
    giota3 = lax.broadcasted_iota(jnp.int32, (1, 1, G), 2)
    dist_ref[...] = jnp.full((FPS_BH, R, LANES), 1e10, jnp.float32)

    def step(k, carry):
        far, civ = carry
        ds = []
        for b in range(FPS_BH):
            farb = jnp.sum(jnp.where(biota3 == b, far, 0))
            r = farb // LANES
            c = farb % LANES
            cm = lane1 == c
            db = jnp.zeros((R, LANES), jnp.float32)
            for ch in range(3):
                row = xyz_ref[b, ch, pl.ds(r, 1), :]
                cch = jnp.sum(jnp.where(cm, row, 0.0))
                diff = xyz_ref[b, ch] - cch
                db = db + diff * diff
            ds.append(db[None])
        d = jnp.concatenate(ds, axis=0)
        nd = jnp.minimum(dist_ref[...], d)
        dist_ref[...] = nd
        civ = jnp.where(giota3 == k, far, civ)
        mval = jnp.max(nd, axis=(1, 2), keepdims=True)
        nxt = jnp.min(jnp.where(nd == mval, pid3, N),
                      axis=(1, 2), keepdims=True).astype(jnp.int32)
        return nxt, civ

    _, civ = lax.fori_loop(
        0, G, step,
        (jnp.zeros((FPS_BH, 1, 1), jnp.int32),
         jnp.zeros((FPS_BH, 1, G), jnp.int32)))
    cidx_ref[...] = civ


def _run_fps(xyz4):
    return pl.pallas_call(
        _fps_body,
        grid=(B // FPS_BH,),
        in_specs=[pl.BlockSpec((FPS_BH, 3, R, LANES), lambda b: (b, 0, 0, 0))],
        out_specs=pl.BlockSpec((FPS_BH, 1, G), lambda b: (b, 0, 0)),
        out_shape=jax.ShapeDtypeStruct((B, 1, G), jnp.int32),
        scratch_shapes=[pltpu.VMEM((FPS_BH, R, LANES), jnp.float32)],
    )(xyz4)


def _knn_body(xyz_ref, cidx_ref, knn_ref, cen_ref):
    rowi = lax.broadcasted_iota(jnp.int32, (R, LANES), 0)
    lanei = lax.broadcasted_iota(jnp.int32, (R, LANES), 1)
    pid = rowi * LANES + lanei
    lane1 = lax.broadcasted_iota(jnp.int32, (1, LANES), 1)
    giota = lax.broadcasted_iota(jnp.int32, (1, G), 1)
    miota = lax.broadcasted_iota(jnp.int32, (1, M), 1)

    x0 = xyz_ref[0, 0]
    x1 = xyz_ref[0, 1]
    x2 = xyz_ref[0, 2]
    psq = x0 * x0 + x1 * x1 + x2 * x2
    x0b = x0.astype(jnp.bfloat16).astype(jnp.float32)
    x1b = x1.astype(jnp.bfloat16).astype(jnp.float32)
    x2b = x2.astype(jnp.bfloat16).astype(jnp.float32)

    pid3 = pid[None]
    miota3 = lax.broadcasted_iota(jnp.int32, (1, 1, M), 2)
    GC = 8

    def g_body(gc, _):
        d2s = []
        for i in range(GC):
            g = gc * GC + i
            cidx_g = jnp.sum(jnp.where(giota == g, cidx_ref[0], 0))
            r = cidx_g // LANES
            c = cidx_g % LANES
            cm = lane1 == c
            cs = []
            for ch in range(3):
                row = xyz_ref[0, ch, pl.ds(r, 1), :]
                cs.append(jnp.sum(jnp.where(cm, row, 0.0)))
            c0, c1, c2 = cs
            c0b = c0.astype(jnp.bfloat16).astype(jnp.float32)
            c1b = c1.astype(jnp.bfloat16).astype(jnp.float32)
            c2b = c2.astype(jnp.bfloat16).astype(jnp.float32)
            dot = c0b * x0b + c1b * x1b + c2b * x2b
            csq = c0 * c0 + c1 * c1 + c2 * c2
            d2s.append(((csq + psq) - 2.0 * dot)[None])
            cen_ref[0, pl.ds(g, 1), :] = jnp.concatenate(
                [c0.reshape(1, 1), c1.reshape(1, 1), c2.reshape(1, 1)], axis=1)
        d2 = jnp.concatenate(d2s, axis=0)

        def k_body(k, carry):
            d, rowm = carry
            mv = jnp.min(d, axis=(1, 2), keepdims=True)
            pk = jnp.min(jnp.where(d == mv, pid3, N),
                         axis=(1, 2), keepdims=True).astype(jnp.int32)
            rowm = jnp.where(miota3 == k, pk, rowm)
            d = jnp.where(pid3 == pk, BIG, d)
            return d, rowm

        _, rowm = lax.fori_loop(0, M, k_body,
                                (d2, jnp.zeros((GC, 1, M), jnp.int32)))
        knn_ref[0, pl.ds(gc * GC, GC), :] = rowm.reshape(GC, M)
        return 0

    lax.fori_loop(0, G // GC, g_body, 0)


def _run_knn(xyz4, cidx):
    return pl.pallas_call(
        _knn_body,
        grid=(B,),
        in_specs=[
            pl.BlockSpec((1, 3, R, LANES), lambda b: (b, 0, 0, 0)),
            pl.BlockSpec((1, 1, G), lambda b: (b, 0, 0)),
        ],
        out_specs=[
            pl.BlockSpec((1, G, M), lambda b: (b, 0, 0)),
            pl.BlockSpec((1, G, 3), lambda b: (b, 0, 0)),
        ],
        out_shape=[
            jax.ShapeDtypeStruct((B, G, M), jnp.int32),
            jax.ShapeDtypeStruct((B, G, 3), jnp.float32),
        ],
    )(xyz4, cidx)


SC_CHUNK = BG * IN_C
SC_T = 24
BP = B * 3 * N


def _sc_gather_body(table_hbm, knnt_hbm, out_hbm, knn_v, idx_v, rows_v, sem):
    info = plsc.get_sparse_core_info()
    nc = info.num_cores
    wid = lax.axis_index("s") * nc + lax.axis_index("c")
    pltpu.sync_copy(knnt_hbm.at[pl.ds(wid, 1)], knn_v)
    for t in range(SC_T):
        ch = t // 4
        q = t % 4
        for s in range(8):
            bg0 = q * 128 + s * 16
            b = bg0 // G
            if ch < 3:
                base = (b * 3 + ch) * N
            else:
                base = BP + (b * 3 + (ch - 3)) * N
            kv = knn_v[0, pl.ds(bg0, 16)]
            idx_v[t, pl.ds(s * 16, 16)] = kv + base
    cps = []
    for t in range(SC_T):
        cps.append(pltpu.async_copy(table_hbm.at[idx_v.at[t]], rows_v.at[t], sem))
    for cp in cps:
        cp.wait()
    pltpu.sync_copy(rows_v, out_hbm.at[wid])


def _run_sc_gather(table, knn_t):
    mesh = plsc.VectorSubcoreMesh(core_axis_name="c", subcore_axis_name="s")
    k = functools.partial(
        pl.kernel,
        mesh=mesh,
        out_type=jax.ShapeDtypeStruct((32, SC_T, 128), jnp.float32),
        scratch_types=[
            pltpu.VMEM((1, BG), jnp.int32),
            pltpu.VMEM((SC_T, 128), jnp.int32),
            pltpu.VMEM((SC_T, 128), jnp.float32),
            pltpu.SemaphoreType.DMA,
        ],
    )(_sc_gather_body)
    return k(table, knn_t)


def _enc_body(x_ref, cen_ref, w1_ref, b1_ref, g1_ref, be1_ref, w2_ref, b2_ref,
              w3a_ref, w3b_ref, b3_ref, g3_ref, be3_ref, w4_ref, b4_ref,
              out_ref, f2_ref):
    w1 = w1_ref[...]
    npts = jnp.float32(M * BG)
    dn_t = (((0,), (0,)), ((), ()))

    def a_m(m):
        xm = x_ref[pl.ds(m, 1)][0] - cen_ref[...]
        return lax.dot_general(xm, w1, dn_t,
                               preferred_element_type=jnp.float32) + b1_ref[...]

    s1 = lax.fori_loop(
        0, M, lambda m, s: s + jnp.sum(a_m(m), axis=0, keepdims=True),
        jnp.zeros((1, 128), jnp.float32))
    mu1 = s1 / npts

    def q1_body(m, q):
        d = a_m(m) - mu1
        return q + jnp.sum(d * d, axis=0, keepdims=True)

    q1 = lax.fori_loop(0, M, q1_body, jnp.zeros((1, 128), jnp.float32))
    inv1 = g1_ref[...] / jnp.sqrt(q1 / npts + EPS)

    w2 = w2_ref[...]

    def l2_body(m, fg):
        h = jnp.maximum((a_m(m) - mu1) * inv1 + be1_ref[...], 0.0)
        f2 = jnp.dot(h, w2, preferred_element_type=jnp.float32) + b2_ref[...]
        f2_ref[pl.ds(m, 1)] = f2[None]
        return jnp.maximum(fg, f2)

    fg = lax.fori_loop(0, M, l2_body, jnp.full((BG, 256), -BIG, jnp.float32))

    gpart = jnp.dot(fg, w3a_ref[...], preferred_element_type=jnp.float32) + b3_ref[...]
    w3b = w3b_ref[...]

    def f3_m(m):
        return gpart + jnp.dot(f2_ref[pl.ds(m, 1)][0], w3b,
                               preferred_element_type=jnp.float32)

    s3 = lax.fori_loop(
        0, M, lambda m, s: s + jnp.sum(f3_m(m), axis=0, keepdims=True),
        jnp.zeros((1, 512), jnp.float32))
    mu3 = s3 / npts

    def q3_body(m, q):
        d = f3_m(m) - mu3
        return q + jnp.sum(d * d, axis=0, keepdims=True)

    q3 = lax.fori_loop(0, M, q3_body, jnp.zeros((1, 512), jnp.float32))
    inv3 = g3_ref[...] / jnp.sqrt(q3 / npts + EPS)

    w4 = w4_ref[...]

    def l4_body(m, acc):
        h = jnp.maximum((f3_m(m) - mu3) * inv3 + be3_ref[...], 0.0)
        f4 = jnp.dot(h, w4, preferred_element_type=jnp.float32) + b4_ref[...]
        return jnp.maximum(acc, f4)

    acc = lax.fori_loop(0, M, l4_body, jnp.full((BG, HID), -BIG, jnp.float32))
    out_ref[...] = acc


def _run_encoder(x, cen6, w1t, b1, g1, be1, w2t, b2, w3at, w3bt, b3, g3, be3,
                 w4t, b4):
    return pl.pallas_call(
        _enc_body,
        out_shape=jax.ShapeDtypeStruct((BG, HID), jnp.float32),
        scratch_shapes=[
            pltpu.VMEM((M, BG, 256), jnp.float32),
        ],
    )(x, cen6, w1t, b1, g1, be1, w2t, b2, w3at, w3bt, b3, g3, be3, w4t, b4)


def kernel(pcd_pos, pcd_rgb, pcd_mask, W1, b1, g1, be1, W2, b2, W3, b3, g3,
           be3, W4, b4):
    del pcd_mask
    xyz4 = pcd_pos.reshape(B, 3, R, LANES)
    cidx = _run_fps(xyz4)
    knn, centers = _run_knn(xyz4, cidx)

    table = jnp.concatenate([pcd_pos.reshape(-1), pcd_rgb.reshape(-1)])
    knn_t = knn.reshape(BG, M).T
    xg = _run_sc_gather(table, knn_t)
    x = xg.reshape(M, IN_C, BG)
    cen_t = jnp.pad(centers.reshape(BG, 3).T, ((0, 3), (0, 0)))

    out = _run_encoder(
        x, cen_t,
        W1.T,
        b1.reshape(1, 128), g1.reshape(1, 128), be1.reshape(1, 128),
        W2.T, b2.reshape(1, 256),
        W3[:, :256].T, W3[:, 256:].T, b3.reshape(1, 512),
        g3.reshape(1, 512), be3.reshape(1, 512),
        W4.T, b4.reshape(1, HID),
    )
    return out.reshape(B, G, HID)
